# Initial kernel scaffold; baseline (speedup 1.0000x reference)
import jax
import jax.numpy as jnp
from jax.experimental import pallas as pl


def kernel(x, edge_index, edge_attr, Wn, bn, We, be, t, W1, b1, g1, bg1, W2, b2, gn, bn2, Wl, bl):
    raise NotImplementedError("write your pallas kernel here")


# speedup vs baseline: 1.8820x; 1.8820x over previous
import functools

import jax
import jax.numpy as jnp
from jax import lax
from jax.experimental import pallas as pl
from jax.experimental.pallas import tpu as pltpu
from jax.experimental.pallas import tpu_sc as plsc

N = 10000
E = 320000
D_IN = 8
H = 256
L = 7
OUT = 112
EPS = 1e-7

NW = 32
NPT = 320
NPAD = NW * NPT
HC = 128
B = 128
EP = E + 2 * B


def _sc_edge_body(h2, ea2, idxh, idxe, dstp, e0s, e1s, Cm, tsp, out,
                  bih, bie, bdst, gih, gie, rh, re, cv, tv, b0v, b1v,
                  num, den, sem1, sem2):
    wid = lax.axis_index("s") * 2 + lax.axis_index("c")
    nbase = wid * NPT

    pltpu.sync_copy(e0s.at[wid], b0v)
    pltpu.sync_copy(e1s.at[wid], b1v)
    pltpu.sync_copy(tsp, tv)
    e0 = b0v[...][0]
    e1 = b1v[...][0]
    a0 = e0 - lax.rem(e0, 8)
    nb = (e1 - a0 + B - 1) // B

    tvec = tv[...]
    cols = [lax.iota(jnp.int32, 16) + 16 * k for k in range(8)]

    for c in range(2):
        pltpu.sync_copy(Cm.at[c], cv)
        cvk = [cv[pl.ds(16 * k, 16)] for k in range(8)]

        def zero_body(i, carry):
            z16 = jnp.zeros((16,), jnp.float32)
            for k in range(8):
                num[i, pl.ds(16 * k, 16)] = z16
                den[i, pl.ds(16 * k, 16)] = z16
            return carry
        lax.fori_loop(0, NPT, zero_body, 0)

        def batch_body(b, carry):
            abase = pl.multiple_of(a0 + b * B, 8)
            pltpu.sync_copy(idxh.at[pl.ds(abase, B)], bih)
            pltpu.sync_copy(idxe.at[pl.ds(abase, B)], bie)
            pltpu.sync_copy(dstp.at[pl.ds(abase, B)], bdst)
            for k in range(8):
                gih[pl.ds(16 * k, 16)] = bih[pl.ds(16 * k, 16)] + c
                gie[pl.ds(16 * k, 16)] = bie[pl.ds(16 * k, 16)] + c
            cp1 = pltpu.async_copy(h2.at[gih], rh, sem1)
            cp2 = pltpu.async_copy(ea2.at[gie], re, sem2)
            cp1.wait()
            cp2.wait()
            jstart = jnp.maximum(e0 - abase, 0)
            jend = jnp.minimum(e1 - abase, B)
            for g in range(8):
                glo = jnp.clip(jstart - 16 * g, 0, 16)
                ghi = jnp.clip(jend - 16 * g, 0, 16)
                dv = bdst[pl.ds(16 * g, 16)] - nbase

                def edge_body(i, ec, g=g, dv=dv):
                    row = jnp.take_along_axis(
                        dv, jnp.full((16,), i, jnp.int32), axis=0,
                        mode="promise_in_bounds")
                    j = 16 * g + i
                    for k in range(8):
                        hv = rh[j, pl.ds(16 * k, 16)]
                        ev = re[j, pl.ds(16 * k, 16)]
                        m = jnp.maximum(hv + ev, 0.0) + EPS
                        s = m * tvec - cvk[k]
                        ez = jnp.exp(s)
                        plsc.addupdate_scatter(num, [row, cols[k]], m * ez)
                        plsc.addupdate_scatter(den, [row, cols[k]], ez)
                    return ec
                lax.fori_loop(glo, ghi, edge_body, 0)
            return carry
        lax.fori_loop(0, nb, batch_body, 0)

        def div_body(i, carry):
            for k in range(8):
                nv = num[i, pl.ds(16 * k, 16)]
                dnv = den[i, pl.ds(16 * k, 16)]
                num[i, pl.ds(16 * k, 16)] = jnp.where(dnv > 0.0, nv / dnv, 0.0)
            return carry
        lax.fori_loop(0, NPT, div_body, 0)

        pltpu.sync_copy(num, out.at[pl.ds(nbase, NPT), pl.ds(HC * c, HC)])


_sc_edge = functools.partial(
    pl.kernel,
    out_type=jax.ShapeDtypeStruct((NPAD, H), jnp.float32),
    mesh=plsc.VectorSubcoreMesh(core_axis_name="c", subcore_axis_name="s"),
    compiler_params=pltpu.CompilerParams(needs_layout_passes=False),
    scratch_types=[
        pltpu.VMEM((B,), jnp.int32),
        pltpu.VMEM((B,), jnp.int32),
        pltpu.VMEM((B,), jnp.int32),
        pltpu.VMEM((B,), jnp.int32),
        pltpu.VMEM((B,), jnp.int32),
        pltpu.VMEM((B, HC), jnp.float32),
        pltpu.VMEM((B, HC), jnp.float32),
        pltpu.VMEM((HC,), jnp.float32),
        pltpu.VMEM((16,), jnp.float32),
        pltpu.VMEM((16,), jnp.int32),
        pltpu.VMEM((16,), jnp.int32),
        pltpu.VMEM((NPT, HC), jnp.float32),
        pltpu.VMEM((NPT, HC), jnp.float32),
        pltpu.SemaphoreType.DMA,
        pltpu.SemaphoreType.DMA,
    ],
)(_sc_edge_body)


def _enc_body(xb, wb, bb, ob, mb):
    o = jnp.dot(xb[...], wb[...], preferred_element_type=jnp.float32) + bb[...]
    ob[...] = o
    pm = jnp.max(o, axis=0, keepdims=True)

    @pl.when(pl.program_id(0) == 0)
    def _():
        mb[...] = pm

    @pl.when(pl.program_id(0) != 0)
    def _():
        mb[...] = jnp.maximum(mb[...], pm)


def _encode(xin, w, b, rows_per_block):
    m, d = xin.shape
    grid = m // rows_per_block
    return pl.pallas_call(
        _enc_body,
        grid=(grid,),
        in_specs=[
            pl.BlockSpec((rows_per_block, d), lambda i: (i, 0)),
            pl.BlockSpec((d, H), lambda i: (0, 0)),
            pl.BlockSpec((1, H), lambda i: (0, 0)),
        ],
        out_specs=[
            pl.BlockSpec((rows_per_block, H), lambda i: (i, 0)),
            pl.BlockSpec((1, H), lambda i: (0, 0)),
        ],
        out_shape=[
            jax.ShapeDtypeStruct((m, H), jnp.float32),
            jax.ShapeDtypeStruct((1, H), jnp.float32),
        ],
    )(xin, w, b.reshape(1, H))


def _ln_rows(u, g, bv):
    mu = jnp.mean(u, axis=-1, keepdims=True)
    var = jnp.mean((u - mu) ** 2, axis=-1, keepdims=True)
    return (u - mu) / jnp.sqrt(var + 1e-5) * g + bv


def _mlp_body(hb, ab, w1b, b1b, g1b, bg1b, w2b, b2b, gnb, bn2b, ob, mb):
    h = hb[...]
    hin = ab[...] + h
    u = jnp.dot(hin, w1b[...], preferred_element_type=jnp.float32) + b1b[...]
    u = jax.nn.relu(_ln_rows(u, g1b[...], bg1b[...]))
    v = jnp.dot(u, w2b[...], preferred_element_type=jnp.float32) + b2b[...]
    cc = jax.nn.relu(_ln_rows(v, gnb[...], bn2b[...]))
    hn = h + cc
    ob[...] = hn
    pm = jnp.max(hn, axis=0, keepdims=True)

    @pl.when(pl.program_id(0) == 0)
    def _():
        mb[...] = pm

    @pl.when(pl.program_id(0) != 0)
    def _():
        mb[...] = jnp.maximum(mb[...], pm)


def _mlp(h, agg, w1, b1, g1, bg1, w2, b2, gnv, bn2v):
    R = 400
    grid = N // R
    return pl.pallas_call(
        _mlp_body,
        grid=(grid,),
        in_specs=[
            pl.BlockSpec((R, H), lambda i: (i, 0)),
            pl.BlockSpec((R, H), lambda i: (i, 0)),
            pl.BlockSpec((H, 2 * H), lambda i: (0, 0)),
            pl.BlockSpec((1, 2 * H), lambda i: (0, 0)),
            pl.BlockSpec((1, 2 * H), lambda i: (0, 0)),
            pl.BlockSpec((1, 2 * H), lambda i: (0, 0)),
            pl.BlockSpec((2 * H, H), lambda i: (0, 0)),
            pl.BlockSpec((1, H), lambda i: (0, 0)),
            pl.BlockSpec((1, H), lambda i: (0, 0)),
            pl.BlockSpec((1, H), lambda i: (0, 0)),
        ],
        out_specs=[
            pl.BlockSpec((R, H), lambda i: (i, 0)),
            pl.BlockSpec((1, H), lambda i: (0, 0)),
        ],
        out_shape=[
            jax.ShapeDtypeStruct((N, H), jnp.float32),
            jax.ShapeDtypeStruct((1, H), jnp.float32),
        ],
    )(h, agg, w1, b1.reshape(1, 2 * H), g1.reshape(1, 2 * H),
      bg1.reshape(1, 2 * H), w2, b2.reshape(1, H), gnv.reshape(1, H),
      bn2v.reshape(1, H))


def _head_body(hb, wb, bb, ob):
    ob[...] = jnp.dot(hb[...], wb[...], preferred_element_type=jnp.float32) + bb[...]


def _head(h, wl, bl):
    R = 2000
    grid = N // R
    return pl.pallas_call(
        _head_body,
        grid=(grid,),
        in_specs=[
            pl.BlockSpec((R, H), lambda i: (i, 0)),
            pl.BlockSpec((H, OUT), lambda i: (0, 0)),
            pl.BlockSpec((1, OUT), lambda i: (0, 0)),
        ],
        out_specs=pl.BlockSpec((R, OUT), lambda i: (i, 0)),
        out_shape=jax.ShapeDtypeStruct((N, OUT), jnp.float32),
    )(h, wl, bl.reshape(1, OUT))


def kernel(x, edge_index, edge_attr, Wn, bn, We, be, t, W1, b1, g1, bg1,
           W2, b2, gn, bn2, Wl, bl):
    src = edge_index[0]
    dst = edge_index[1]
    order = jnp.argsort(dst).astype(jnp.int32)
    dst_s = jnp.take(dst, order).astype(jnp.int32)
    src_s = jnp.take(src, order).astype(jnp.int32)
    idxh = jnp.pad(src_s * 2, (0, EP - E))
    idxe = jnp.pad(order * 2, (0, EP - E))
    dstp = jnp.pad(dst_s, (0, EP - E))
    bounds = (jnp.arange(33, dtype=jnp.int32) * NPT).astype(jnp.int32)
    starts = jnp.searchsorted(dst_s, bounds).astype(jnp.int32)
    e0s = jnp.broadcast_to(starts[:32, None], (32, 16))
    e1s = jnp.broadcast_to(starts[1:33, None], (32, 16))

    h, hmax = _encode(x, Wn, bn, 2000)
    ea, eamax = _encode(edge_attr, We, be, 4000)
    ea2 = ea.reshape(2 * E, HC)

    for i in range(L):
        Cm = (t[i] * (hmax + eamax + EPS)).reshape(2, HC)
        tsp = jnp.full((16,), t[i], jnp.float32)
        agg = _sc_edge(h.reshape(2 * N, HC), ea2, idxh, idxe, dstp,
                       e0s, e1s, Cm, tsp)
        h, hmax = _mlp(h, agg, W1[i], b1[i], g1[i], bg1[i], W2[i], b2[i],
                       gn[i], bn2[i])
    return _head(h, Wl, bl)
